# baseline (speedup 1.0000x reference)
import functools

import jax
import jax.numpy as jnp
from jax import lax
from jax.experimental import pallas as pl
from jax.experimental.pallas import tpu as pltpu
from jax.experimental.pallas import tpu_sc as plsc

_NC = 2
_NS = 16
_NW = _NC * _NS
_L = 16


def _sc_edge_pass(table, src, dst):
    n = table.shape[1]
    e = src.shape[0]
    epw = e // _NW
    steps = epw // _L

    def body(tbl_hbm, src_hbm, dst_hbm, out_hbm, tbl_v, acc_v, src_v, dst_v, sem):
        wid = lax.axis_index("s") * _NC + lax.axis_index("c")
        base = wid * epw
        cp_t = pltpu.async_copy(tbl_hbm, tbl_v, sem)
        cp_s = pltpu.async_copy(src_hbm.at[pl.ds(base, epw)], src_v, sem)
        cp_d = pltpu.async_copy(dst_hbm.at[pl.ds(base, epw)], dst_v, sem)

        zero = jnp.zeros((_L,), jnp.float32)

        @plsc.parallel_loop(0, 5 * n, step=_L, unroll=8)
        def _zero(off):
            acc_v[pl.ds(off, _L)] = zero

        cp_t.wait()
        cp_s.wait()
        cp_d.wait()

        rows = [jnp.full((_L,), r * n, jnp.int32) for r in range(6)]

        @plsc.parallel_loop(0, epw, step=_L, unroll=8)
        def _edges(off):
            sidx = src_v[pl.ds(off, _L)]
            didx = dst_v[pl.ds(off, _L)]
            av = plsc.load_gather(tbl_v, [sidx + rows[4]])
            dv = plsc.load_gather(tbl_v, [didx + rows[5]])
            s = av + dv
            w = jnp.exp(jnp.maximum(s, 0.2 * s))
            plsc.addupdate_scatter(acc_v, [didx + rows[4]], w)
            for r in range(4):
                h = plsc.load_gather(tbl_v, [sidx + rows[r]])
                plsc.addupdate_scatter(acc_v, [didx + rows[r]], h * w)

        pltpu.sync_copy(acc_v, out_hbm.at[wid])

    return pl.kernel(
        body,
        out_type=jax.ShapeDtypeStruct((_NW, 5 * n), jnp.float32),
        mesh=plsc.VectorSubcoreMesh(
            core_axis_name="c", subcore_axis_name="s",
            num_cores=_NC, num_subcores=_NS),
        compiler_params=pltpu.CompilerParams(needs_layout_passes=False),
        scratch_types=[
            pltpu.VMEM((6 * n,), jnp.float32),
            pltpu.VMEM((5 * n,), jnp.float32),
            pltpu.VMEM((epw,), jnp.int32),
            pltpu.VMEM((epw,), jnp.int32),
            pltpu.SemaphoreType.DMA,
        ],
    )(table.reshape(-1), src, dst).reshape(_NW, 5, n)


def _identity4():
    r = lax.broadcasted_iota(jnp.int32, (4, 4), 0)
    c = lax.broadcasted_iota(jnp.int32, (4, 4), 1)
    return (r == c).astype(jnp.float32)


def _channel_attention_t(o, w1, b1c, w2, b2c):
    i4 = _identity4()
    m = jnp.sum(o, axis=1, keepdims=True) * (1.0 / o.shape[1])
    s_row = jnp.sum(m * w1, axis=0, keepdims=True)
    s_col = jnp.sum(s_row * i4, axis=1, keepdims=True)
    s_col = jnp.maximum(s_col + b1c, 0.0)
    g_row = jnp.sum(s_col * w2, axis=0, keepdims=True)
    g_col = jnp.sum(g_row * i4, axis=1, keepdims=True) + b2c
    return o * (1.0 / (1.0 + jnp.exp(-g_col)))


def _tc_pre(x, pt, a_s, a_d, res_b):
    n = x.shape[0]

    def body(x_ref, pt_ref, as_ref, ad_ref, rb_ref, tbl_ref, res_ref):
        y = lax.dot_general(pt_ref[...], x_ref[...], (((1,), (1,)), ((), ())),
                            preferred_element_type=jnp.float32)
        h = y[0:4]
        asr = jnp.sum(h * as_ref[...], axis=0, keepdims=True)
        adr = jnp.sum(h * ad_ref[...], axis=0, keepdims=True)
        tbl_ref[...] = jnp.concatenate([h, asr, adr], axis=0)
        res_ref[...] = y[4:8] + rb_ref[...]

    return pl.pallas_call(
        body,
        out_shape=(jax.ShapeDtypeStruct((6, n), jnp.float32),
                   jax.ShapeDtypeStruct((4, n), jnp.float32)),
    )(x, pt, a_s, a_d, res_b)


def _reduce_norm(acc_ref, bias_c):
    a = acc_ref[0]
    for i in range(1, _NW):
        a = a + acc_ref[i]
    o = a[0:4] / (a[4:5] + 1e-16) + bias_c
    return jnp.maximum(o, 0.0)


def _tc_mid(acc, b1c, cw1, cb1c, cw2, cb2c, w2t, as2, ad2):
    n = acc.shape[2]

    def body(acc_ref, b1_ref, w1_ref, bb1_ref, w2_ref, bb2_ref, w2t_ref,
             as_ref, ad_ref, tbl_ref):
        o = _reduce_norm(acc_ref, b1_ref[...])
        hca = _channel_attention_t(o, w1_ref[...], bb1_ref[...],
                                   w2_ref[...], bb2_ref[...])
        h2 = lax.dot_general(w2t_ref[...], hca, (((1,), (0,)), ((), ())),
                             preferred_element_type=jnp.float32)
        asr = jnp.sum(h2 * as_ref[...], axis=0, keepdims=True)
        adr = jnp.sum(h2 * ad_ref[...], axis=0, keepdims=True)
        tbl_ref[...] = jnp.concatenate([h2, asr, adr], axis=0)

    return pl.pallas_call(
        body,
        out_shape=jax.ShapeDtypeStruct((6, n), jnp.float32),
    )(acc, b1c, cw1, cb1c, cw2, cb2c, w2t, as2, ad2)


def _tc_post(acc, b2c, cw1, cb1c, cw2, cb2c, resid, fc_c, fcb):
    n = acc.shape[2]

    def body(acc_ref, b2_ref, w1_ref, bb1_ref, w2_ref, bb2_ref, res_ref,
             fc_ref, fcb_ref, out_ref):
        o = _reduce_norm(acc_ref, b2_ref[...])
        hca = _channel_attention_t(o, w1_ref[...], bb1_ref[...],
                                   w2_ref[...], bb2_ref[...])
        f = hca + res_ref[...]
        logit = jnp.sum(f * fc_ref[...], axis=0, keepdims=True) + fcb_ref[...]
        out_ref[...] = 1.0 / (1.0 + jnp.exp(-logit))

    return pl.pallas_call(
        body,
        out_shape=jax.ShapeDtypeStruct((1, n), jnp.float32),
    )(acc, b2c, cw1, cb1c, cw2, cb2c, resid, fc_c, fcb)


def kernel(x, edge_index, W1, a_src1, a_dst1, b1, ca1_w1, ca1_b1, ca1_w2,
           ca1_b2, W2, a_src2, a_dst2, b2, ca2_w1, ca2_b1, ca2_w2, ca2_b2,
           res_W, res_b, fc_W, fc_b):
    src = edge_index[0]
    dst = edge_index[1]
    pt = jnp.concatenate([W1.T, res_W.T], axis=0)

    tbl1, resid = _tc_pre(x, pt, a_src1.reshape(4, 1), a_dst1.reshape(4, 1),
                          res_b.reshape(4, 1))
    acc1 = _sc_edge_pass(tbl1, src, dst)
    tbl2 = _tc_mid(acc1, b1.reshape(4, 1), ca1_w1, ca1_b1.reshape(4, 1),
                   ca1_w2, ca1_b2.reshape(4, 1), W2.T,
                   a_src2.reshape(4, 1), a_dst2.reshape(4, 1))
    acc2 = _sc_edge_pass(tbl2, src, dst)
    out = _tc_post(acc2, b2.reshape(4, 1), ca2_w1, ca2_b1.reshape(4, 1),
                   ca2_w2, ca2_b2.reshape(4, 1), resid,
                   fc_W.reshape(4, 1), fc_b.reshape(1, 1))
    return out.reshape(-1, 1)


# speedup vs baseline: 1.2464x; 1.2464x over previous
import jax
import jax.numpy as jnp
from jax import lax
from jax.experimental import pallas as pl
from jax.experimental.pallas import tpu as pltpu
from jax.experimental.pallas import tpu_sc as plsc

_NC = 2
_NS = 16
_NW = _NC * _NS
_L = 16


def _sc_edge_pass(table, src2d, dst2d):
    n = table.shape[0] // 6
    e = src2d.shape[1]
    epw = (e // _NW) // 128 * 128
    nx = (e - epw * _NW) // 128
    ph = epw // 3
    buf = 2 * ph + 128
    blk = ((5 * n) + (16 * 128 - 1)) // (16 * 128) * 128
    apad = 16 * blk

    def body(tbl_hbm, src_hbm, dst_hbm, out_hbm, scr_hbm, tbl_v, acc_v,
             src_v, dst_v, sem):
        cid = lax.axis_index("c")
        sid = lax.axis_index("s")
        wid = sid * _NC + cid
        base = wid * epw
        xoff = epw * _NW + jnp.minimum(wid, nx - 1) * 128
        cp_t = pltpu.async_copy(tbl_hbm, tbl_v.at[pl.ds(0, 6 * n)], sem)
        cp_s = pltpu.async_copy(src_hbm.at[0, pl.ds(base, ph)],
                                src_v.at[pl.ds(0, ph)], sem)
        cp_d = pltpu.async_copy(dst_hbm.at[0, pl.ds(base, ph)],
                                dst_v.at[pl.ds(0, ph)], sem)

        zero = jnp.zeros((_L,), jnp.float32)

        @plsc.parallel_loop(0, apad, step=_L, unroll=8)
        def _zfill(off):
            acc_v[pl.ds(off, _L)] = zero

        cp_t.wait()
        cp_s.wait()
        cp_d.wait()

        rows = [jnp.full((_L,), r * n, jnp.int32) for r in range(6)]

        def edge_step(off):
            sidx = src_v[pl.ds(off, _L)]
            didx = dst_v[pl.ds(off, _L)]
            av = plsc.load_gather(tbl_v, [sidx + rows[4]])
            dv = plsc.load_gather(tbl_v, [didx + rows[5]])
            s = av + dv
            w = jnp.exp(jnp.maximum(s, 0.2 * s))
            plsc.addupdate_scatter(acc_v, [didx + rows[4]], w)
            for r in range(4):
                h = plsc.load_gather(tbl_v, [sidx + rows[r]])
                plsc.addupdate_scatter(acc_v, [didx + rows[r]], h * w)

        cp_s2 = pltpu.async_copy(src_hbm.at[0, pl.ds(base + ph, ph)],
                                 src_v.at[pl.ds(ph, ph)], sem)
        cp_d2 = pltpu.async_copy(dst_hbm.at[0, pl.ds(base + ph, ph)],
                                 dst_v.at[pl.ds(ph, ph)], sem)
        plsc.parallel_loop(0, ph, step=_L, unroll=4)(edge_step)
        cp_s2.wait()
        cp_d2.wait()
        cp_s3 = pltpu.async_copy(src_hbm.at[0, pl.ds(base + 2 * ph, ph)],
                                 src_v.at[pl.ds(0, ph)], sem)
        cp_d3 = pltpu.async_copy(dst_hbm.at[0, pl.ds(base + 2 * ph, ph)],
                                 dst_v.at[pl.ds(0, ph)], sem)
        cp_sx = pltpu.async_copy(src_hbm.at[0, pl.ds(xoff, 128)],
                                 src_v.at[pl.ds(2 * ph, 128)], sem)
        cp_dx = pltpu.async_copy(dst_hbm.at[0, pl.ds(xoff, 128)],
                                 dst_v.at[pl.ds(2 * ph, 128)], sem)
        plsc.parallel_loop(ph, 2 * ph, step=_L, unroll=4)(edge_step)
        cp_s3.wait()
        cp_d3.wait()
        cp_sx.wait()
        cp_dx.wait()
        plsc.parallel_loop(0, ph, step=_L, unroll=4)(edge_step)

        @pl.when(wid < nx)
        def _extra():
            plsc.parallel_loop(2 * ph, 2 * ph + 128, step=_L, unroll=4)(edge_step)

        pltpu.sync_copy(acc_v, scr_hbm.at[wid])
        plsc.subcore_barrier()
        cps = [
            pltpu.async_copy(
                scr_hbm.at[slab * _NC + cid, pl.ds(sid * blk, blk)],
                tbl_v.at[pl.ds(slab * blk, blk)], sem)
            for slab in range(16)
        ]
        for cp in cps:
            cp.wait()

        @plsc.parallel_loop(0, blk, step=_L, unroll=2)
        def _reduce(off):
            v = tbl_v[pl.ds(off, _L)]
            for slab in range(1, 16):
                v = v + tbl_v[pl.ds(slab * blk + off, _L)]
            acc_v[pl.ds(off, _L)] = v

        pltpu.sync_copy(acc_v.at[pl.ds(0, blk)],
                        out_hbm.at[cid, pl.ds(sid * blk, blk)])

    return pl.kernel(
        body,
        out_type=(jax.ShapeDtypeStruct((_NC, apad), jnp.float32),
                  jax.ShapeDtypeStruct((_NW, apad), jnp.float32)),
        mesh=plsc.VectorSubcoreMesh(
            core_axis_name="c", subcore_axis_name="s",
            num_cores=_NC, num_subcores=_NS),
        compiler_params=pltpu.CompilerParams(needs_layout_passes=False),
        scratch_types=[
            pltpu.VMEM((max(6 * n, 16 * blk),), jnp.float32),
            pltpu.VMEM((apad,), jnp.float32),
            pltpu.VMEM((buf,), jnp.int32),
            pltpu.VMEM((buf,), jnp.int32),
            pltpu.SemaphoreType.DMA,
        ],
    )(table, src2d, dst2d)[0]


def _identity4():
    r = lax.broadcasted_iota(jnp.int32, (4, 4), 0)
    c = lax.broadcasted_iota(jnp.int32, (4, 4), 1)
    return (r == c).astype(jnp.float32)


def _col(row4):
    return jnp.sum(row4 * _identity4(), axis=1, keepdims=True)


def _channel_attention_t(o, w1, b1r, w2, b2r, nvalid):
    m = jnp.sum(o, axis=1, keepdims=True) * (1.0 / nvalid)
    s_row = jnp.sum(m * w1, axis=0, keepdims=True)
    s_col = jnp.maximum(_col(s_row + b1r), 0.0)
    g_row = jnp.sum(s_col * w2, axis=0, keepdims=True)
    g_col = _col(g_row + b2r)
    return o * (1.0 / (1.0 + jnp.exp(-g_col)))


def _tc_pre(x, ei, w1, a_s, a_d, res_w, res_b):
    n = x.shape[0]
    e = ei.shape[1]

    def body(x_ref, ei_ref, w1_ref, as_ref, ad_ref, rw_ref, rb_ref,
             tbl_ref, res_ref, src_ref, dst_ref):
        xv = x_ref[...]
        h = lax.dot_general(w1_ref[...], xv, (((0,), (1,)), ((), ())),
                            preferred_element_type=jnp.float32)
        asr = jnp.sum(h * _col(as_ref[...]), axis=0, keepdims=True)
        adr = jnp.sum(h * _col(ad_ref[...]), axis=0, keepdims=True)
        tbl_ref[...] = jnp.concatenate([h, asr, adr], axis=0)
        rv = lax.dot_general(rw_ref[...], xv, (((0,), (1,)), ((), ())),
                             preferred_element_type=jnp.float32)
        res_ref[...] = rv + _col(rb_ref[...])
        eiv = ei_ref[...]
        src_ref[...] = eiv[0:1]
        dst_ref[...] = eiv[1:2]

    return pl.pallas_call(
        body,
        out_shape=(jax.ShapeDtypeStruct((6, n), jnp.float32),
                   jax.ShapeDtypeStruct((4, n), jnp.float32),
                   jax.ShapeDtypeStruct((1, e), jnp.int32),
                   jax.ShapeDtypeStruct((1, e), jnp.int32)),
    )(x, ei, w1, a_s, a_d, res_w, res_b)


def _reduce_norm(acc_ref, bias_row):
    a = acc_ref[0] + acc_ref[1]
    o = a[0:4] / (a[4:5] + 1e-16) + _col(bias_row)
    return jnp.maximum(o, 0.0)


def _tc_mid(acc, b1, cw1, cb1, cw2, cb2, w2, as2, ad2):
    n = acc.shape[2]

    def body(acc_ref, b1_ref, w1_ref, bb1_ref, w2_ref, bb2_ref, gw2_ref,
             as_ref, ad_ref, tbl_ref):
        o = _reduce_norm(acc_ref, b1_ref[...])
        hca = _channel_attention_t(o, w1_ref[...], bb1_ref[...],
                                   w2_ref[...], bb2_ref[...], n)
        h2 = lax.dot_general(gw2_ref[...], hca, (((0,), (0,)), ((), ())),
                             preferred_element_type=jnp.float32)
        asr = jnp.sum(h2 * _col(as_ref[...]), axis=0, keepdims=True)
        adr = jnp.sum(h2 * _col(ad_ref[...]), axis=0, keepdims=True)
        tbl_ref[...] = jnp.concatenate([h2, asr, adr], axis=0)

    return pl.pallas_call(
        body,
        out_shape=jax.ShapeDtypeStruct((6, n), jnp.float32),
    )(acc, b1, cw1, cb1, cw2, cb2, w2, as2, ad2)


def _tc_post(acc, b2, cw1, cb1, cw2, cb2, resid, fc_c, fcb):
    n = acc.shape[2]

    def body(acc_ref, b2_ref, w1_ref, bb1_ref, w2_ref, bb2_ref, res_ref,
             fc_ref, fcb_ref, out_ref):
        o = _reduce_norm(acc_ref, b2_ref[...])
        hca = _channel_attention_t(o, w1_ref[...], bb1_ref[...],
                                   w2_ref[...], bb2_ref[...], n)
        f = hca + res_ref[...]
        logit = jnp.sum(f * fc_ref[...], axis=0, keepdims=True) + fcb_ref[...]
        out_ref[...] = 1.0 / (1.0 + jnp.exp(-logit))

    return pl.pallas_call(
        body,
        out_shape=jax.ShapeDtypeStruct((1, n), jnp.float32),
    )(acc, b2, cw1, cb1, cw2, cb2, resid, fc_c, fcb)


def kernel(x, edge_index, W1, a_src1, a_dst1, b1, ca1_w1, ca1_b1, ca1_w2,
           ca1_b2, W2, a_src2, a_dst2, b2, ca2_w1, ca2_b1, ca2_w2, ca2_b2,
           res_W, res_b, fc_W, fc_b):
    n = x.shape[0]

    def row(v):
        return v.reshape(1, 4)

    tbl1, resid, src2d, dst2d = _tc_pre(x, edge_index, W1, row(a_src1),
                                        row(a_dst1), res_W, row(res_b))
    acc1 = _sc_edge_pass(tbl1.reshape(-1), src2d, dst2d)
    acc1 = acc1[:, :5 * n].reshape(_NC, 5, n)
    tbl2 = _tc_mid(acc1, row(b1), ca1_w1, row(ca1_b1), ca1_w2, row(ca1_b2),
                   W2, row(a_src2), row(a_dst2))
    acc2 = _sc_edge_pass(tbl2.reshape(-1), src2d, dst2d)
    acc2 = acc2[:, :5 * n].reshape(_NC, 5, n)
    out = _tc_post(acc2, row(b2), ca2_w1, row(ca2_b1), ca2_w2, row(ca2_b2),
                   resid, fc_W, fcb=fc_b.reshape(1, 1))
    return out.reshape(-1, 1)


# speedup vs baseline: 1.3508x; 1.0838x over previous
import jax
import jax.numpy as jnp
from jax import lax
from jax.experimental import pallas as pl
from jax.experimental.pallas import tpu as pltpu
from jax.experimental.pallas import tpu_sc as plsc

_NC = 2
_NS = 16
_NW = _NC * _NS
_L = 16


def _sc_edge_pass(table, src2d, dst2d, nvalid):
    st = table.shape[1] // 6
    n = nvalid
    e = src2d.shape[1]
    epw = (e // _NW) // 128 * 128
    nx = (e - epw * _NW) // 128
    ph = epw // 3
    buf = 2 * ph + 128
    blk = (5 * st + 16 * 128 - 1) // (16 * 128) * 128
    apad = 16 * blk

    def body(tbl_hbm, src_hbm, dst_hbm, out_hbm, scr_hbm, tbl_v, acc_v,
             src_v, dst_v, sem):
        cid = lax.axis_index("c")
        sid = lax.axis_index("s")
        wid = sid * _NC + cid
        base = wid * epw
        xoff = epw * _NW + jnp.minimum(wid, nx - 1) * 128
        cp_t = pltpu.async_copy(tbl_hbm.at[0, pl.ds(0, 6 * st)],
                                tbl_v.at[pl.ds(0, 6 * st)], sem)
        cp_s = pltpu.async_copy(src_hbm.at[0, pl.ds(base, ph)],
                                src_v.at[pl.ds(0, ph)], sem)
        cp_d = pltpu.async_copy(dst_hbm.at[0, pl.ds(base, ph)],
                                dst_v.at[pl.ds(0, ph)], sem)

        zero = jnp.zeros((_L,), jnp.float32)

        @plsc.parallel_loop(0, apad, step=_L, unroll=8)
        def _zfill(off):
            acc_v[pl.ds(off, _L)] = zero

        cp_t.wait()
        cp_s.wait()
        cp_d.wait()

        rows = [jnp.full((_L,), r * st, jnp.int32) for r in range(6)]

        def edge_step(off):
            sidx = src_v[pl.ds(off, _L)]
            didx = dst_v[pl.ds(off, _L)]
            av = plsc.load_gather(tbl_v, [sidx + rows[4]])
            dv = plsc.load_gather(tbl_v, [didx + rows[5]])
            s = av + dv
            w = jnp.exp(jnp.maximum(s, 0.2 * s))
            plsc.addupdate_scatter(acc_v, [didx + rows[4]], w)
            for r in range(4):
                h = plsc.load_gather(tbl_v, [sidx + rows[r]])
                plsc.addupdate_scatter(acc_v, [didx + rows[r]], h * w)

        cp_s2 = pltpu.async_copy(src_hbm.at[0, pl.ds(base + ph, ph)],
                                 src_v.at[pl.ds(ph, ph)], sem)
        cp_d2 = pltpu.async_copy(dst_hbm.at[0, pl.ds(base + ph, ph)],
                                 dst_v.at[pl.ds(ph, ph)], sem)
        plsc.parallel_loop(0, ph, step=_L, unroll=4)(edge_step)
        cp_s2.wait()
        cp_d2.wait()
        cp_s3 = pltpu.async_copy(src_hbm.at[0, pl.ds(base + 2 * ph, ph)],
                                 src_v.at[pl.ds(0, ph)], sem)
        cp_d3 = pltpu.async_copy(dst_hbm.at[0, pl.ds(base + 2 * ph, ph)],
                                 dst_v.at[pl.ds(0, ph)], sem)
        cp_sx = pltpu.async_copy(src_hbm.at[0, pl.ds(xoff, 128)],
                                 src_v.at[pl.ds(2 * ph, 128)], sem)
        cp_dx = pltpu.async_copy(dst_hbm.at[0, pl.ds(xoff, 128)],
                                 dst_v.at[pl.ds(2 * ph, 128)], sem)
        plsc.parallel_loop(ph, 2 * ph, step=_L, unroll=4)(edge_step)
        cp_s3.wait()
        cp_d3.wait()
        cp_sx.wait()
        cp_dx.wait()
        plsc.parallel_loop(0, ph, step=_L, unroll=4)(edge_step)

        @pl.when(wid < nx)
        def _extra():
            plsc.parallel_loop(2 * ph, 2 * ph + 128, step=_L, unroll=4)(edge_step)

        pltpu.sync_copy(acc_v, scr_hbm.at[wid])
        plsc.subcore_barrier()
        cps = [
            pltpu.async_copy(
                scr_hbm.at[slab * _NC + cid, pl.ds(sid * blk, blk)],
                tbl_v.at[pl.ds(slab * blk, blk)], sem)
            for slab in range(16)
        ]
        for cp in cps:
            cp.wait()

        @plsc.parallel_loop(0, blk, step=_L, unroll=2)
        def _reduce(off):
            v = tbl_v[pl.ds(off, _L)]
            for slab in range(1, 16):
                v = v + tbl_v[pl.ds(slab * blk + off, _L)]
            acc_v[pl.ds(off, _L)] = v

        pltpu.sync_copy(acc_v.at[pl.ds(0, blk)],
                        out_hbm.at[0, pl.ds(cid * apad + sid * blk, blk)])

    return pl.kernel(
        body,
        out_type=(jax.ShapeDtypeStruct((1, _NC * apad), jnp.float32),
                  jax.ShapeDtypeStruct((_NW, apad), jnp.float32)),
        mesh=plsc.VectorSubcoreMesh(
            core_axis_name="c", subcore_axis_name="s",
            num_cores=_NC, num_subcores=_NS),
        compiler_params=pltpu.CompilerParams(needs_layout_passes=False),
        scratch_types=[
            pltpu.VMEM((6 * st,), jnp.float32),
            pltpu.VMEM((apad,), jnp.float32),
            pltpu.VMEM((buf,), jnp.int32),
            pltpu.VMEM((buf,), jnp.int32),
            pltpu.SemaphoreType.DMA,
        ],
    )(table, src2d, dst2d)[0]


def _identity4():
    r = lax.broadcasted_iota(jnp.int32, (4, 4), 0)
    c = lax.broadcasted_iota(jnp.int32, (4, 4), 1)
    return (r == c).astype(jnp.float32)


def _col(row4):
    return jnp.sum(row4 * _identity4(), axis=1, keepdims=True)


def _channel_attention_t(o, w1, b1r, w2, b2r, nvalid):
    m = jnp.sum(o, axis=1, keepdims=True) * (1.0 / nvalid)
    s_row = jnp.sum(m * w1, axis=0, keepdims=True)
    s_col = jnp.maximum(_col(s_row + b1r), 0.0)
    g_row = jnp.sum(s_col * w2, axis=0, keepdims=True)
    g_col = _col(g_row + b2r)
    return o * (1.0 / (1.0 + jnp.exp(-g_col)))


def _tc_pre(x, ei, w1, a_s, a_d, res_w, res_b):
    n = x.shape[0]
    e = ei.shape[1]

    st = (n + 127) // 128 * 128

    def body(x_ref, ei_ref, w1_ref, as_ref, ad_ref, rw_ref, rb_ref,
             tbl_ref, res_ref, src_ref, dst_ref):
        xv = x_ref[...]
        h = lax.dot_general(w1_ref[...], xv, (((0,), (1,)), ((), ())),
                            preferred_element_type=jnp.float32)
        asr = jnp.sum(h * _col(as_ref[...]), axis=0, keepdims=True)
        adr = jnp.sum(h * _col(ad_ref[...]), axis=0, keepdims=True)
        _store_table(tbl_ref, h, asr, adr, n, st)
        rv = lax.dot_general(rw_ref[...], xv, (((0,), (1,)), ((), ())),
                             preferred_element_type=jnp.float32)
        res_ref[...] = rv + _col(rb_ref[...])
        eiv = ei_ref[...]
        src_ref[...] = eiv[0:1]
        dst_ref[...] = eiv[1:2]

    return pl.pallas_call(
        body,
        out_shape=(jax.ShapeDtypeStruct((1, 6 * st), jnp.float32),
                   jax.ShapeDtypeStruct((4, n), jnp.float32),
                   jax.ShapeDtypeStruct((1, e), jnp.int32),
                   jax.ShapeDtypeStruct((1, e), jnp.int32)),
    )(x, ei, w1, a_s, a_d, res_w, res_b)


def _reduce_norm(acc_ref, bias_row, n, st):
    apad = (5 * st + 16 * 128 - 1) // (16 * 128) * 128 * 16
    rws = [acc_ref[0:1, pl.ds(r * st, n)] + acc_ref[0:1, pl.ds(apad + r * st, n)]
           for r in range(5)]
    num = jnp.concatenate(rws[0:4], axis=0)
    o = num / (rws[4] + 1e-16) + _col(bias_row)
    return jnp.maximum(o, 0.0)


def _store_table(tbl_ref, h, asr, adr, n, st):
    for r in range(4):
        tbl_ref[0:1, pl.ds(r * st, n)] = h[r:r + 1]
    tbl_ref[0:1, pl.ds(4 * st, n)] = asr
    tbl_ref[0:1, pl.ds(5 * st, n)] = adr


def _tc_mid(acc, b1, cw1, cb1, cw2, cb2, w2, as2, ad2, n):
    st = (n + 127) // 128 * 128

    def body(acc_ref, b1_ref, w1_ref, bb1_ref, w2_ref, bb2_ref, gw2_ref,
             as_ref, ad_ref, tbl_ref):
        o = _reduce_norm(acc_ref, b1_ref[...], n, st)
        hca = _channel_attention_t(o, w1_ref[...], bb1_ref[...],
                                   w2_ref[...], bb2_ref[...], n)
        h2 = lax.dot_general(gw2_ref[...], hca, (((0,), (0,)), ((), ())),
                             preferred_element_type=jnp.float32)
        asr = jnp.sum(h2 * _col(as_ref[...]), axis=0, keepdims=True)
        adr = jnp.sum(h2 * _col(ad_ref[...]), axis=0, keepdims=True)
        _store_table(tbl_ref, h2, asr, adr, n, st)

    return pl.pallas_call(
        body,
        out_shape=jax.ShapeDtypeStruct((1, 6 * st), jnp.float32),
    )(acc, b1, cw1, cb1, cw2, cb2, w2, as2, ad2)


def _tc_post(acc, b2, cw1, cb1, cw2, cb2, resid, fc_c, fcb, n):
    st = (n + 127) // 128 * 128

    def body(acc_ref, b2_ref, w1_ref, bb1_ref, w2_ref, bb2_ref, res_ref,
             fc_ref, fcb_ref, out_ref):
        o = _reduce_norm(acc_ref, b2_ref[...], n, st)
        hca = _channel_attention_t(o, w1_ref[...], bb1_ref[...],
                                   w2_ref[...], bb2_ref[...], n)
        f = hca + res_ref[...]
        logit = jnp.sum(f * fc_ref[...], axis=0, keepdims=True) + fcb_ref[...]
        out_ref[...] = 1.0 / (1.0 + jnp.exp(-logit))

    return pl.pallas_call(
        body,
        out_shape=jax.ShapeDtypeStruct((1, n), jnp.float32),
    )(acc, b2, cw1, cb1, cw2, cb2, resid, fc_c, fcb)


def kernel(x, edge_index, W1, a_src1, a_dst1, b1, ca1_w1, ca1_b1, ca1_w2,
           ca1_b2, W2, a_src2, a_dst2, b2, ca2_w1, ca2_b1, ca2_w2, ca2_b2,
           res_W, res_b, fc_W, fc_b):
    n = x.shape[0]

    def row(v):
        return v.reshape(1, 4)

    tbl1, resid, src2d, dst2d = _tc_pre(x, edge_index, W1, row(a_src1),
                                        row(a_dst1), res_W, row(res_b))
    acc1 = _sc_edge_pass(tbl1, src2d, dst2d, n)
    tbl2 = _tc_mid(acc1, row(b1), ca1_w1, row(ca1_b1), ca1_w2, row(ca1_b2),
                   W2, row(a_src2), row(a_dst2), n)
    acc2 = _sc_edge_pass(tbl2, src2d, dst2d, n)
    out = _tc_post(acc2, row(b2), ca2_w1, row(ca2_b1), ca2_w2, row(ca2_b2),
                   resid, fc_W, fc_b.reshape(1, 1), n)
    return out.reshape(-1, 1)


# speedup vs baseline: 1.5558x; 1.1518x over previous
import jax
import jax.numpy as jnp
from jax import lax
from jax.experimental import pallas as pl
from jax.experimental.pallas import tpu as pltpu
from jax.experimental.pallas import tpu_sc as plsc

_NC = 2
_NS = 16
_NW = _NC * _NS
_L = 16


def _sc_edge_pass(table, src2d, dst2d, nvalid):
    st = table.shape[1] // 4
    n = nvalid
    e = src2d.shape[1]
    epw = (e // _NW) // 128 * 128
    nx = (e - epw * _NW) // 128
    ph = epw // 3
    buf = 2 * ph + 128
    blk = (5 * st + 16 * 128 - 1) // (16 * 128) * 128
    apad = 16 * blk

    def body(tbl_hbm, src_hbm, dst_hbm, out_hbm, scr_hbm, tbl_v, acc_v,
             src_v, dst_v, sem):
        cid = lax.axis_index("c")
        sid = lax.axis_index("s")
        wid = sid * _NC + cid
        base = wid * epw
        xoff = epw * _NW + jnp.minimum(wid, nx - 1) * 128
        cp_t = pltpu.async_copy(tbl_hbm.at[0, pl.ds(0, 4 * st)],
                                tbl_v.at[pl.ds(0, 4 * st)], sem)
        cp_s = pltpu.async_copy(src_hbm.at[0, pl.ds(base, ph)],
                                src_v.at[pl.ds(0, ph)], sem)
        cp_d = pltpu.async_copy(dst_hbm.at[0, pl.ds(base, ph)],
                                dst_v.at[pl.ds(0, ph)], sem)

        zero = jnp.zeros((_L,), jnp.float32)

        @plsc.parallel_loop(0, apad, step=_L, unroll=8)
        def _zfill(off):
            acc_v[pl.ds(off, _L)] = zero

        cp_t.wait()
        cp_s.wait()
        cp_d.wait()

        rows = [jnp.full((_L,), r * st, jnp.int32) for r in range(5)]
        himask = jnp.full((_L,), -65536, jnp.int32)

        def edge_step(off):
            sidx = src_v[pl.ds(off, _L)]
            didx = dst_v[pl.ds(off, _L)]
            av = plsc.load_gather(tbl_v, [sidx + rows[2]])
            dv = plsc.load_gather(tbl_v, [didx + rows[3]])
            s = av + dv
            w = jnp.exp(jnp.maximum(s, 0.2 * s))
            plsc.addupdate_scatter(acc_v, [didx + rows[4]], w)
            for q in range(2):
                hp = plsc.load_gather(tbl_v, [sidx + rows[q]])
                u = plsc.bitcast(hp, jnp.int32)
                hlo = plsc.bitcast(u << 16, jnp.float32)
                hhi = plsc.bitcast(u & himask, jnp.float32)
                plsc.addupdate_scatter(acc_v, [didx + rows[2 * q]], hlo * w)
                plsc.addupdate_scatter(acc_v, [didx + rows[2 * q + 1]], hhi * w)

        cp_s2 = pltpu.async_copy(src_hbm.at[0, pl.ds(base + ph, ph)],
                                 src_v.at[pl.ds(ph, ph)], sem)
        cp_d2 = pltpu.async_copy(dst_hbm.at[0, pl.ds(base + ph, ph)],
                                 dst_v.at[pl.ds(ph, ph)], sem)
        plsc.parallel_loop(0, ph, step=_L, unroll=4)(edge_step)
        cp_s2.wait()
        cp_d2.wait()
        cp_s3 = pltpu.async_copy(src_hbm.at[0, pl.ds(base + 2 * ph, ph)],
                                 src_v.at[pl.ds(0, ph)], sem)
        cp_d3 = pltpu.async_copy(dst_hbm.at[0, pl.ds(base + 2 * ph, ph)],
                                 dst_v.at[pl.ds(0, ph)], sem)
        cp_sx = pltpu.async_copy(src_hbm.at[0, pl.ds(xoff, 128)],
                                 src_v.at[pl.ds(2 * ph, 128)], sem)
        cp_dx = pltpu.async_copy(dst_hbm.at[0, pl.ds(xoff, 128)],
                                 dst_v.at[pl.ds(2 * ph, 128)], sem)
        plsc.parallel_loop(ph, 2 * ph, step=_L, unroll=4)(edge_step)
        cp_s3.wait()
        cp_d3.wait()
        cp_sx.wait()
        cp_dx.wait()
        plsc.parallel_loop(0, ph, step=_L, unroll=4)(edge_step)

        @pl.when(wid < nx)
        def _extra():
            plsc.parallel_loop(2 * ph, 2 * ph + 128, step=_L, unroll=4)(edge_step)

        pltpu.sync_copy(acc_v, scr_hbm.at[wid])
        plsc.subcore_barrier()
        cps = [
            pltpu.async_copy(
                scr_hbm.at[slab * _NC + cid, pl.ds(sid * blk, blk)],
                tbl_v.at[pl.ds(slab * blk, blk)], sem)
            for slab in range(16)
        ]
        for cp in cps:
            cp.wait()

        @plsc.parallel_loop(0, blk, step=_L, unroll=2)
        def _reduce(off):
            v = tbl_v[pl.ds(off, _L)]
            for slab in range(1, 16):
                v = v + tbl_v[pl.ds(slab * blk + off, _L)]
            acc_v[pl.ds(off, _L)] = v

        pltpu.sync_copy(acc_v.at[pl.ds(0, blk)],
                        out_hbm.at[0, pl.ds(cid * apad + sid * blk, blk)])

    return pl.kernel(
        body,
        out_type=(jax.ShapeDtypeStruct((1, _NC * apad), jnp.float32),
                  jax.ShapeDtypeStruct((_NW, apad), jnp.float32)),
        mesh=plsc.VectorSubcoreMesh(
            core_axis_name="c", subcore_axis_name="s",
            num_cores=_NC, num_subcores=_NS),
        compiler_params=pltpu.CompilerParams(needs_layout_passes=False),
        scratch_types=[
            pltpu.VMEM((max(4 * st, 16 * blk),), jnp.float32),
            pltpu.VMEM((apad,), jnp.float32),
            pltpu.VMEM((buf,), jnp.int32),
            pltpu.VMEM((buf,), jnp.int32),
            pltpu.SemaphoreType.DMA,
        ],
    )(table, src2d, dst2d)[0]


def _identity4():
    r = lax.broadcasted_iota(jnp.int32, (4, 4), 0)
    c = lax.broadcasted_iota(jnp.int32, (4, 4), 1)
    return (r == c).astype(jnp.float32)


def _col(row4):
    return jnp.sum(row4 * _identity4(), axis=1, keepdims=True)


def _channel_attention_t(o, w1, b1r, w2, b2r, nvalid):
    m = jnp.sum(o, axis=1, keepdims=True) * (1.0 / nvalid)
    s_row = jnp.sum(m * w1, axis=0, keepdims=True)
    s_col = jnp.maximum(_col(s_row + b1r), 0.0)
    g_row = jnp.sum(s_col * w2, axis=0, keepdims=True)
    g_col = _col(g_row + b2r)
    return o * (1.0 / (1.0 + jnp.exp(-g_col)))


def _tc_pre(x, ei, w1, a_s, a_d, res_w, res_b):
    n = x.shape[0]
    e = ei.shape[1]

    st = (n + 127) // 128 * 128

    def body(x_ref, ei_ref, w1_ref, as_ref, ad_ref, rw_ref, rb_ref,
             tbl_ref, res_ref, src_ref, dst_ref):
        xv = x_ref[...]
        h = lax.dot_general(w1_ref[...], xv, (((0,), (1,)), ((), ())),
                            preferred_element_type=jnp.float32)
        asr = jnp.sum(h * _col(as_ref[...]), axis=0, keepdims=True)
        adr = jnp.sum(h * _col(ad_ref[...]), axis=0, keepdims=True)
        _store_table(tbl_ref, h, asr, adr, n, st)
        rv = lax.dot_general(rw_ref[...], xv, (((0,), (1,)), ((), ())),
                             preferred_element_type=jnp.float32)
        res_ref[...] = rv + _col(rb_ref[...])
        eiv = ei_ref[...]
        src_ref[...] = eiv[0:1]
        dst_ref[...] = eiv[1:2]

    return pl.pallas_call(
        body,
        out_shape=(jax.ShapeDtypeStruct((1, 4 * st), jnp.float32),
                   jax.ShapeDtypeStruct((4, n), jnp.float32),
                   jax.ShapeDtypeStruct((1, e), jnp.int32),
                   jax.ShapeDtypeStruct((1, e), jnp.int32)),
    )(x, ei, w1, a_s, a_d, res_w, res_b)


def _reduce_norm(acc_ref, bias_row, n, st):
    apad = (5 * st + 16 * 128 - 1) // (16 * 128) * 128 * 16
    rws = [acc_ref[0:1, pl.ds(r * st, n)] + acc_ref[0:1, pl.ds(apad + r * st, n)]
           for r in range(5)]
    num = jnp.concatenate(rws[0:4], axis=0)
    o = num / (rws[4] + 1e-16) + _col(bias_row)
    return jnp.maximum(o, 0.0)


def _store_table(tbl_ref, h, asr, adr, n, st):
    hu = lax.bitcast_convert_type(
        lax.convert_element_type(h, jnp.bfloat16), jnp.uint16)
    hu = lax.convert_element_type(hu, jnp.uint32)
    for q in range(2):
        u = (hu[2 * q + 1:2 * q + 2] << 16) | hu[2 * q:2 * q + 1]
        tbl_ref[0:1, pl.ds(q * st, n)] = lax.bitcast_convert_type(u, jnp.float32)
    tbl_ref[0:1, pl.ds(2 * st, n)] = asr
    tbl_ref[0:1, pl.ds(3 * st, n)] = adr


def _tc_mid(acc, b1, cw1, cb1, cw2, cb2, w2, as2, ad2, n):
    st = (n + 127) // 128 * 128

    def body(acc_ref, b1_ref, w1_ref, bb1_ref, w2_ref, bb2_ref, gw2_ref,
             as_ref, ad_ref, tbl_ref):
        o = _reduce_norm(acc_ref, b1_ref[...], n, st)
        hca = _channel_attention_t(o, w1_ref[...], bb1_ref[...],
                                   w2_ref[...], bb2_ref[...], n)
        h2 = lax.dot_general(gw2_ref[...], hca, (((0,), (0,)), ((), ())),
                             preferred_element_type=jnp.float32)
        asr = jnp.sum(h2 * _col(as_ref[...]), axis=0, keepdims=True)
        adr = jnp.sum(h2 * _col(ad_ref[...]), axis=0, keepdims=True)
        _store_table(tbl_ref, h2, asr, adr, n, st)

    return pl.pallas_call(
        body,
        out_shape=jax.ShapeDtypeStruct((1, 4 * st), jnp.float32),
    )(acc, b1, cw1, cb1, cw2, cb2, w2, as2, ad2)


def _tc_post(acc, b2, cw1, cb1, cw2, cb2, resid, fc_c, fcb, n):
    st = (n + 127) // 128 * 128

    def body(acc_ref, b2_ref, w1_ref, bb1_ref, w2_ref, bb2_ref, res_ref,
             fc_ref, fcb_ref, out_ref):
        o = _reduce_norm(acc_ref, b2_ref[...], n, st)
        hca = _channel_attention_t(o, w1_ref[...], bb1_ref[...],
                                   w2_ref[...], bb2_ref[...], n)
        f = hca + res_ref[...]
        logit = jnp.sum(f * fc_ref[...], axis=0, keepdims=True) + fcb_ref[...]
        out_ref[...] = 1.0 / (1.0 + jnp.exp(-logit))

    return pl.pallas_call(
        body,
        out_shape=jax.ShapeDtypeStruct((1, n), jnp.float32),
    )(acc, b2, cw1, cb1, cw2, cb2, resid, fc_c, fcb)


def kernel(x, edge_index, W1, a_src1, a_dst1, b1, ca1_w1, ca1_b1, ca1_w2,
           ca1_b2, W2, a_src2, a_dst2, b2, ca2_w1, ca2_b1, ca2_w2, ca2_b2,
           res_W, res_b, fc_W, fc_b):
    n = x.shape[0]

    def row(v):
        return v.reshape(1, 4)

    tbl1, resid, src2d, dst2d = _tc_pre(x, edge_index, W1, row(a_src1),
                                        row(a_dst1), res_W, row(res_b))
    acc1 = _sc_edge_pass(tbl1, src2d, dst2d, n)
    tbl2 = _tc_mid(acc1, row(b1), ca1_w1, row(ca1_b1), ca1_w2, row(ca1_b2),
                   W2, row(a_src2), row(a_dst2), n)
    acc2 = _sc_edge_pass(tbl2, src2d, dst2d, n)
    out = _tc_post(acc2, row(b2), ca2_w1, row(ca2_b1), ca2_w2, row(ca2_b2),
                   resid, fc_W, fc_b.reshape(1, 1), n)
    return out.reshape(-1, 1)
